# baseline (speedup 1.0000x reference)
import jax
import jax.numpy as jnp
from jax import lax
from jax.experimental import pallas as pl
from jax.experimental.pallas import tpu as pltpu
from jax.experimental.pallas import tpu_sc as plsc

E = 8
K = 2
H = 1024
I = 4096
T = 2048
B = 256
NBLK = T * K // B + E
MAXROWS = NBLK * B
CH = 512
NC = I // CH

NCORES = 2
NSUB = 16
NW = NCORES * NSUB
L = 16

_MESH = dict(core_axis_name="c", subcore_axis_name="s",
             num_cores=NCORES, num_subcores=NSUB)
_SC_PARAMS = pltpu.CompilerParams(needs_layout_passes=False)


TPW_A = T // NW
NCH_A = TPW_A // L
NCHG = T // L


def _top2(lbuf, off):
    zero = jnp.zeros((L,), jnp.int32)
    v1 = lbuf[pl.ds(off, L)]
    i1 = zero
    v2 = jnp.full((L,), -jnp.inf, jnp.float32)
    i2 = zero
    for e in range(1, E):
        ve = lbuf[pl.ds(e * T + off, L)]
        gt1 = ve > v1
        gt2 = ve > v2
        i2 = jnp.where(gt1, i1, jnp.where(gt2, jnp.int32(e), i2))
        v2 = jnp.where(gt1, v1, jnp.where(gt2, ve, v2))
        i1 = jnp.where(gt1, jnp.int32(e), i1)
        v1 = jnp.where(gt1, ve, v1)
    return v1, i1, v2, i2


def _route_body(logits_flat, r0_hbm, r1_hbm, w0_hbm, w1_hbm, meta_hbm,
                lbuf, r0b, r1b, w0b, w1b, bebuf):
    cid = lax.axis_index("c")
    sid = lax.axis_index("s")
    wid = sid * NCORES + cid
    iota = lax.iota(jnp.int32, L)
    zero = jnp.zeros((L,), jnp.int32)
    pltpu.sync_copy(logits_flat, lbuf)

    my_first = wid * NCH_A

    @pl.loop(0, NCHG, init_carry=(zero, zero))
    def scan(c, carry):
        totals, myprefix = carry
        myprefix = jnp.where(c == my_first, totals, myprefix)
        _, i1, _, i2 = _top2(lbuf, c * L)
        for e in range(E):
            s = jnp.sum(jnp.where(jnp.logical_or(i1 == e, i2 == e),
                                  jnp.int32(1), jnp.int32(0)))
            totals = totals + jnp.where(iota == e, s, zero)
        return totals, myprefix

    totals, myprefix = scan
    padded = lax.shift_left(lax.shift_right_logical(totals + (B - 1), 8), 8)
    poff = plsc.cumsum(padded) - padded
    nact = lax.shift_right_logical(jnp.sum(padded), 8)

    cnt2 = poff + myprefix
    for c in range(NCH_A):
        off = (my_first + c) * L
        v1, i1, v2, i2 = _top2(lbuf, off)
        w0 = 1.0 / (1.0 + jnp.exp(v2 - v1))
        sl = pl.ds(c * L, L)
        w0b[sl] = w0
        w1b[sl] = 1.0 - w0
        r0 = zero
        r1 = zero
        for e in range(E):
            h1 = i1 == e
            h2 = i2 == e
            hi = jnp.where(jnp.logical_or(h1, h2),
                           jnp.int32(1), jnp.int32(0))
            pos = plsc.cumsum(hi) - hi
            d = pos + cnt2[e]
            r0 = jnp.where(h1, d, r0)
            r1 = jnp.where(h2, d, r1)
            cnt2 = cnt2 + jnp.where(iota == e, jnp.sum(hi), zero)
        r0b[sl] = r0
        r1b[sl] = r1

    base_t = wid * TPW_A
    pltpu.sync_copy(r0b, r0_hbm.at[pl.ds(base_t, TPW_A)])
    pltpu.sync_copy(r1b, r1_hbm.at[pl.ds(base_t, TPW_A)])
    pltpu.sync_copy(w0b, w0_hbm.at[pl.ds(base_t, TPW_A)])
    pltpu.sync_copy(w1b, w1_hbm.at[pl.ds(base_t, TPW_A)])

    @pl.when(wid == 0)
    def _():
        ends = lax.shift_right_logical(poff + padded, 8)
        for bc in range(2):
            lanes = iota + bc * L
            bev = zero
            for e in range(E):
                bev = bev + jnp.where(lanes >= ends[e],
                                      jnp.int32(1), jnp.int32(0))
            bebuf[pl.ds(bc * L, L)] = jnp.minimum(bev, E - 1)
        bebuf[pl.ds(2 * L, L)] = jnp.where(iota == 0, nact, zero)
        pltpu.sync_copy(bebuf, meta_hbm)


def _route_sc(logits_flat):
    return pl.kernel(
        _route_body,
        out_type=(
            jax.ShapeDtypeStruct((T,), jnp.int32),
            jax.ShapeDtypeStruct((T,), jnp.int32),
            jax.ShapeDtypeStruct((T,), jnp.float32),
            jax.ShapeDtypeStruct((T,), jnp.float32),
            jax.ShapeDtypeStruct((3 * L,), jnp.int32),
        ),
        mesh=plsc.VectorSubcoreMesh(**_MESH),
        scratch_types=[
            pltpu.VMEM((E * T,), jnp.float32),
            pltpu.VMEM((TPW_A,), jnp.int32),
            pltpu.VMEM((TPW_A,), jnp.int32),
            pltpu.VMEM((TPW_A,), jnp.float32),
            pltpu.VMEM((TPW_A,), jnp.float32),
            pltpu.VMEM((3 * L,), jnp.int32),
        ],
        compiler_params=_SC_PARAMS,
    )(logits_flat)


TPW_B = T // NW


def _scatter_body(hidden, r0_hbm, r1_hbm, xs_hbm, xbuf, idx, sem):
    cid = lax.axis_index("c")
    sid = lax.axis_index("s")
    base = (sid * NCORES + cid) * TPW_B
    pltpu.sync_copy(hidden.at[pl.ds(base, TPW_B)], xbuf)
    pltpu.sync_copy(r0_hbm.at[pl.ds(base, TPW_B)], idx.at[0])
    pltpu.sync_copy(r1_hbm.at[pl.ds(base, TPW_B)], idx.at[1])
    d0 = pltpu.async_copy(xbuf, xs_hbm.at[idx.at[0]], sem)
    d1 = pltpu.async_copy(xbuf, xs_hbm.at[idx.at[1]], sem)
    d0.wait()
    d1.wait()


def _scatter_sc(x, r0, r1):
    return pl.kernel(
        _scatter_body,
        out_type=jax.ShapeDtypeStruct((MAXROWS, H), jnp.float32),
        mesh=plsc.VectorSubcoreMesh(**_MESH),
        scratch_types=[
            pltpu.VMEM((TPW_B, H), jnp.float32),
            pltpu.VMEM((2, TPW_B), jnp.int32),
            pltpu.SemaphoreType.DMA,
        ],
        compiler_params=_SC_PARAMS,
    )(x, r0, r1)


_WBUF = pl.Buffered(buffer_count=5, use_lookahead=True)
_XBUF = pl.Buffered(buffer_count=3, use_lookahead=True)


def _mlp_outer(meta_ref, x_hbm, w1_hbm, w3_hbm, w2_hbm, out_hbm, acc_ref):
    def body(idx, x_ref, w1_ref, w3_ref, w2_ref, out_ref):
        j, i = idx
        active = i < meta_ref[2 * L]

        @pl.when(active)
        def _():
            x = x_ref[...]
            g = lax.dot_general(x, w1_ref[0], (((1,), (1,)), ((), ())),
                                preferred_element_type=jnp.float32)
            u = lax.dot_general(x, w3_ref[0], (((1,), (1,)), ((), ())),
                                preferred_element_type=jnp.float32)
            h = g * jax.nn.sigmoid(g) * u
            contrib = lax.dot_general(h, w2_ref[0], (((1,), (1,)), ((), ())),
                                      preferred_element_type=jnp.float32)

            @pl.when(j == 0)
            def _():
                acc_ref[pl.ds(i * B, B), :] = contrib

            @pl.when(j > 0)
            def _():
                acc_ref[pl.ds(i * B, B), :] = (acc_ref[pl.ds(i * B, B), :]
                                               + contrib)

        @pl.when(jnp.logical_and(j == NC - 1, active))
        def _():
            out_ref[...] = acc_ref[pl.ds(i * B, B), :]

    pipe = pltpu.emit_pipeline(
        body,
        grid=(NC, NBLK),
        in_specs=[
            pl.BlockSpec((B, H),
                         lambda j, i: (jnp.minimum(i, meta_ref[2 * L] - 1),
                                       0),
                         pipeline_mode=_XBUF),
            pl.BlockSpec((1, CH, H), lambda j, i: (meta_ref[i], j, 0),
                         pipeline_mode=_WBUF),
            pl.BlockSpec((1, CH, H), lambda j, i: (meta_ref[i], j, 0),
                         pipeline_mode=_WBUF),
            pl.BlockSpec((1, H, CH), lambda j, i: (meta_ref[i], 0, j),
                         pipeline_mode=_WBUF),
        ],
        out_specs=[
            pl.BlockSpec((B, H),
                         lambda j, i: (jnp.where(j == NC - 1, i, 0), 0)),
        ],
        _explicit_indices=True,
    )
    pipe(x_hbm, w1_hbm, w3_hbm, w2_hbm, out_hbm)


def _grouped_mlp(x_sorted, W1, W3, W2, meta):
    return pl.pallas_call(
        _mlp_outer,
        in_specs=[
            pl.BlockSpec(memory_space=pltpu.SMEM),
            pl.BlockSpec(memory_space=pl.ANY),
            pl.BlockSpec(memory_space=pl.ANY),
            pl.BlockSpec(memory_space=pl.ANY),
            pl.BlockSpec(memory_space=pl.ANY),
        ],
        out_specs=pl.BlockSpec(memory_space=pl.ANY),
        scratch_shapes=[pltpu.VMEM((MAXROWS, H), jnp.float32)],
        out_shape=jax.ShapeDtypeStruct((MAXROWS, H), jnp.float32),
        compiler_params=pltpu.CompilerParams(
            vmem_limit_bytes=100 * 1024 * 1024),
    )(meta, x_sorted, W1, W3, W2)


_HB = TPW_B // 2


def _combine_body(rows_hbm, r0_hbm, r1_hbm, w0_hbm, w1_hbm, out_hbm,
                  idx0, idx1, wb0, wb1, buf0, buf1, sem):
    cid = lax.axis_index("c")
    sid = lax.axis_index("s")
    base = (sid * NCORES + cid) * TPW_B
    pltpu.sync_copy(r0_hbm.at[pl.ds(base, TPW_B)], idx0)
    pltpu.sync_copy(r1_hbm.at[pl.ds(base, TPW_B)], idx1)
    pltpu.sync_copy(w0_hbm.at[pl.ds(base, TPW_B)], wb0)
    pltpu.sync_copy(w1_hbm.at[pl.ds(base, TPW_B)], wb1)
    zero = jnp.zeros((L,), jnp.int32)
    for half in range(2):
        d0 = pltpu.async_copy(rows_hbm.at[idx0.at[pl.ds(half * _HB, _HB)]],
                              buf0, sem)
        d1 = pltpu.async_copy(rows_hbm.at[idx1.at[pl.ds(half * _HB, _HB)]],
                              buf1, sem)
        d0.wait()
        d1.wait()

        @pl.loop(0, _HB)
        def _(t):
            idxv = zero + (half * _HB + t)
            w0v = plsc.load_gather(wb0, [idxv])
            w1v = plsc.load_gather(wb1, [idxv])
            for l in range(H // L):
                sl = pl.ds(l * L, L)
                buf0[t, sl] = buf0[t, sl] * w0v + buf1[t, sl] * w1v

        pltpu.sync_copy(buf0, out_hbm.at[pl.ds(base + half * _HB, _HB)])


def _combine_sc(out_rows, r0, r1, w0, w1):
    return pl.kernel(
        _combine_body,
        out_type=jax.ShapeDtypeStruct((T, H), jnp.float32),
        mesh=plsc.VectorSubcoreMesh(**_MESH),
        scratch_types=[
            pltpu.VMEM((TPW_B,), jnp.int32),
            pltpu.VMEM((TPW_B,), jnp.int32),
            pltpu.VMEM((TPW_B,), jnp.float32),
            pltpu.VMEM((TPW_B,), jnp.float32),
            pltpu.VMEM((_HB, H), jnp.float32),
            pltpu.VMEM((_HB, H), jnp.float32),
            pltpu.SemaphoreType.DMA,
        ],
        compiler_params=_SC_PARAMS,
    )(out_rows, r0, r1, w0, w1)


def kernel(hidden_states, router_logits, W1, W3, W2):
    x = hidden_states.reshape(-1, H)
    logits_flat = router_logits.T.reshape(-1)
    r0, r1, w0, w1, meta = _route_sc(logits_flat)
    x_sorted = _scatter_sc(x, r0, r1)
    out_rows = _grouped_mlp(x_sorted, W1, W3, W2, meta)
    final = _combine_sc(out_rows, r0, r1, w0, w1)
    return final.reshape(hidden_states.shape)
